# baseline (speedup 1.0000x reference)
import functools

import jax
import jax.numpy as jnp
from jax import lax
from jax.experimental import pallas as pl
from jax.experimental.pallas import tpu as pltpu
from jax.experimental.pallas import tpu_sc as plsc

B = 4
A = 2048
OBS = 64
POS = 3
K = 16
LK = 8
HW = 16
BIN = 16
OBS_H = 128
OBS_E = 64
BH = 64
BL = 32
TH = 64
FD = 64
LH = 64
MD = 64
INTRA = 2
INTER = 2
PH = 128
VH = 128
ACT = 32
BIG = 1e9

ROWS = B * A
RBLK = 512
QBLK = 256
TAB1 = 128
TAB2 = 80
TABF = 128


def _ln(x, g, b):
    m = x.mean(-1, keepdims=True)
    v = ((x - m) ** 2).mean(-1, keepdims=True)
    return (x - m) / jnp.sqrt(v + 1e-5) * g + b


def _full_spec(shape):
    n = len(shape)
    return pl.BlockSpec(shape, lambda *_: (0,) * n)


def _row_spec(shape, blk):
    rest = shape[1:]
    n = len(shape)
    return pl.BlockSpec((blk,) + rest, lambda i: (i,) + (0,) * (n - 1))


_SC_NC = 2
_SC_NS = 16
_SC_NW = _SC_NC * _SC_NS


@functools.lru_cache(maxsize=None)
def _make_sc_gather(M, N, D, dtype):
    b_per_w = M // _SC_NW
    row_bytes = D * dtype.itemsize
    chunk = min(512 if row_bytes <= 384 else 256, b_per_w)
    nch = b_per_w // chunk
    mesh = plsc.VectorSubcoreMesh(core_axis_name="c", subcore_axis_name="s")

    @functools.partial(
        pl.kernel,
        mesh=mesh,
        compiler_params=pltpu.CompilerParams(use_tc_tiling_on_sc=False),
        out_type=jax.ShapeDtypeStruct((M, D), dtype),
        scratch_types=[
            pltpu.VMEM((b_per_w,), jnp.int32),
            pltpu.VMEM((chunk, D), dtype),
            pltpu.VMEM((chunk, D), dtype),
            pltpu.SemaphoreType.DMA,
            pltpu.SemaphoreType.DMA,
        ],
    )
    def gather_k(table_hbm, idx_hbm, out_hbm, idx_v, buf0, buf1, sem0, sem1):
        wid = lax.axis_index("s") * _SC_NC + lax.axis_index("c")
        base = wid * b_per_w
        pltpu.sync_copy(idx_hbm.at[pl.ds(base, b_per_w)], idx_v)
        bufs = (buf0, buf1)
        sems = (sem0, sem1)
        cps = [None, None]
        cps[0] = pltpu.async_copy(
            table_hbm.at[idx_v.at[pl.ds(0, chunk)]], bufs[0], sems[0])
        for c in range(nch):
            if c + 1 < nch:
                cps[(c + 1) % 2] = pltpu.async_copy(
                    table_hbm.at[idx_v.at[pl.ds((c + 1) * chunk, chunk)]],
                    bufs[(c + 1) % 2], sems[(c + 1) % 2])
            cps[c % 2].wait()
            pltpu.sync_copy(bufs[c % 2], out_hbm.at[pl.ds(base + c * chunk, chunk)])

    return gather_k


def _sc_gather(table, idx_flat):
    N, D = table.shape
    (M,) = idx_flat.shape
    return _make_sc_gather(M, N, D, table.dtype)(table, idx_flat)


def _encoder_body(obs_ref, hist_ref, pos_ref,
                  bW0, bb0, bg0, bbe0, bWf, bbf,
                  oW0, ob0, og0, obe0, oW1, ob1, og1, obe1, oWf, obf,
                  fW0, fb0, fg0, fbe0, fWf, fbf,
                  pW0, pb0, pg0, pbe0, pWf, pbf,
                  tW1a, tW1b, tb1, imW, imb,
                  bl_out, af_out, pot_out, opart_out, tab1_out, tab2_out):
    hist = hist_ref[...]
    hmean = jnp.mean(hist, axis=1)
    x = jax.nn.gelu(_ln(hmean @ bW0[...] + bb0[...], bg0[...], bbe0[...]))
    bl = x @ bWf[...] + bbf[...]

    o = obs_ref[...]
    o = jax.nn.gelu(_ln(o @ oW0[...] + ob0[...], og0[...], obe0[...]))
    o = jax.nn.gelu(_ln(o @ oW1[...] + ob1[...], og1[...], obe1[...]))
    oe = o @ oWf[...] + obf[...]

    f_in = jnp.concatenate([oe, bl], axis=-1)
    f = jax.nn.gelu(_ln(f_in @ fW0[...] + fb0[...], fg0[...], fbe0[...]))
    af = f @ fWf[...] + fbf[...]

    p = jax.nn.gelu(_ln(af @ pW0[...] + pb0[...], pg0[...], pbe0[...]))
    pot = p @ pWf[...] + pbf[...]

    bp = bl @ tW1b[...]
    msg1 = jax.nn.gelu(af @ imW[...] + imb[...])
    blk = af.shape[0]
    zeros2 = jnp.zeros((blk, TAB2 - FD - POS), jnp.float32)

    bl_out[...] = bl
    af_out[...] = af
    pot_out[...] = pot
    opart_out[...] = oe @ tW1a[...] + tb1[...]
    tab1_out[...] = jnp.concatenate([bp, msg1], axis=-1).astype(jnp.bfloat16)
    tab2_out[...] = jnp.concatenate([af, pos_ref[...], zeros2], axis=-1)


def _run_encoder(obs2, hist3, pos2, p):
    be = p["behavior"]
    ob = p["obs_encoder"]
    fp = p["feature_proj"]
    lp = p["leader_pot"]
    tr = p["trust"]
    tW1 = tr["layers"][0]["W"]
    w_args = [
        be["layers"][0]["W"], be["layers"][0]["b"][None], be["layers"][0]["g"][None], be["layers"][0]["beta"][None],
        be["Wf"], be["bf"][None],
        ob["layers"][0]["W"], ob["layers"][0]["b"][None], ob["layers"][0]["g"][None], ob["layers"][0]["beta"][None],
        ob["layers"][1]["W"], ob["layers"][1]["b"][None], ob["layers"][1]["g"][None], ob["layers"][1]["beta"][None],
        ob["Wf"], ob["bf"][None],
        fp["layers"][0]["W"], fp["layers"][0]["b"][None], fp["layers"][0]["g"][None], fp["layers"][0]["beta"][None],
        fp["Wf"], fp["bf"][None],
        lp["layers"][0]["W"], lp["layers"][0]["b"][None], lp["layers"][0]["g"][None], lp["layers"][0]["beta"][None],
        lp["Wf"], lp["bf"][None],
        tW1[:OBS_E], tW1[OBS_E:], tr["layers"][0]["b"][None],
        p["intra_msg_W"], p["intra_msg_b"][None],
    ]
    in_specs = [_row_spec((ROWS, OBS), RBLK), _row_spec((ROWS, HW, BIN), RBLK),
                _row_spec((ROWS, POS), RBLK)]
    in_specs += [_full_spec(w.shape) for w in w_args]
    out_shapes = [
        jax.ShapeDtypeStruct((ROWS, BL), jnp.float32),
        jax.ShapeDtypeStruct((ROWS, FD), jnp.float32),
        jax.ShapeDtypeStruct((ROWS, 1), jnp.float32),
        jax.ShapeDtypeStruct((ROWS, TH), jnp.float32),
        jax.ShapeDtypeStruct((ROWS, TAB1), jnp.bfloat16),
        jax.ShapeDtypeStruct((ROWS, TAB2), jnp.float32),
    ]
    out_specs = [_row_spec(s.shape, RBLK) for s in out_shapes]
    return pl.pallas_call(
        _encoder_body,
        grid=(ROWS // RBLK,),
        in_specs=in_specs,
        out_specs=out_specs,
        out_shape=out_shapes,
    )(obs2, hist3, pos2, *w_args)


def _topk_body(pq_ref, pall_ref, pot_ref, idx_out, memb_out, npmax_out, *, k):
    pq = pq_ref[0]
    pall = pall_ref[0]
    potrow = pot_ref[0]
    sq_q = jnp.sum(pq * pq, axis=-1, keepdims=True)
    sq_a = jnp.sum(pall * pall, axis=-1)[None, :]
    cross = jax.lax.dot_general(pq, pall, (((1,), (1,)), ((), ())),
                                preferred_element_type=jnp.float32)
    d2 = sq_q + sq_a - 2.0 * cross
    d = jnp.clip(d2, 0.0) + 1e-12
    b = pl.program_id(0)
    i = pl.program_id(1)
    rowid = i * pq.shape[0] + jax.lax.broadcasted_iota(jnp.int32, d.shape, 0)
    colid = jax.lax.broadcasted_iota(jnp.int32, d.shape, 1)
    d = jnp.where(rowid == colid, BIG * BIG, d)
    vals = []
    idxs = []
    selmask = jnp.zeros(d.shape, jnp.bool_)
    for _ in range(k):
        mval = jnp.min(d, axis=1, keepdims=True)
        sel = jnp.min(jnp.where(d == mval, colid, A), axis=1, keepdims=True)
        vals.append(mval)
        idxs.append(sel)
        hit = colid == sel
        selmask = selmask | hit
        d = jnp.where(hit, BIG * BIG, d)
    dvals = jnp.sqrt(jnp.concatenate(vals, axis=1))
    npmax = jnp.max(jnp.where(selmask, potrow, -BIG), axis=1, keepdims=True)
    idx_out[0] = jnp.concatenate(idxs, axis=1) + b * A
    memb_out[0] = jax.nn.softmax(-dvals / 1.0, axis=-1)
    npmax_out[0] = npmax


def _run_topk(positions, pot2):
    body = functools.partial(_topk_body, k=K)
    out_shapes = [
        jax.ShapeDtypeStruct((B, A, K), jnp.int32),
        jax.ShapeDtypeStruct((B, A, K), jnp.float32),
        jax.ShapeDtypeStruct((B, A, 1), jnp.float32),
    ]
    spec_q = pl.BlockSpec((1, QBLK, POS), lambda b, i: (b, i, 0))
    spec_all = pl.BlockSpec((1, A, POS), lambda b, i: (b, 0, 0))
    spec_pot = pl.BlockSpec((1, 1, A), lambda b, i: (b, 0, 0))
    out_spec = pl.BlockSpec((1, QBLK, K), lambda b, i: (b, i, 0))
    out_spec_n = pl.BlockSpec((1, QBLK, 1), lambda b, i: (b, i, 0))
    return pl.pallas_call(
        body,
        grid=(B, A // QBLK),
        in_specs=[spec_q, spec_all, spec_pot],
        out_specs=[out_spec, out_spec, out_spec_n],
        out_shape=out_shapes,
    )(positions, positions, pot2.reshape(B, 1, A))


def _ltopk_body(pq_ref, pall_ref, padq_ref, padall_ref, lw_out, idx_out, *, k):
    pq = pq_ref[0]
    pall = pall_ref[0]
    padq = padq_ref[0][0]
    padall = padall_ref[0][0]
    sq_q = jnp.sum(pq * pq, axis=-1, keepdims=True)
    sq_a = jnp.sum(pall * pall, axis=-1)[None, :]
    cross = jax.lax.dot_general(pq, pall, (((1,), (1,)), ((), ())),
                                preferred_element_type=jnp.float32)
    d2 = sq_q + sq_a - 2.0 * cross
    d = jnp.clip(d2, 0.0) + 1e-12
    b = pl.program_id(0)
    i = pl.program_id(1)
    rowid = i * pq.shape[0] + jax.lax.broadcasted_iota(jnp.int32, d.shape, 0)
    colid = jax.lax.broadcasted_iota(jnp.int32, d.shape, 1)
    valid = (padq[:, None] > 0.5) & (padall[None, :] > 0.5) & (rowid != colid)
    d = jnp.where(valid, d, BIG * BIG)
    vals = []
    idxs = []
    for _ in range(k):
        mval = jnp.min(d, axis=1, keepdims=True)
        sel = jnp.min(jnp.where(d == mval, colid, A), axis=1, keepdims=True)
        vals.append(mval)
        idxs.append(sel)
        d = jnp.where(colid == sel, BIG * BIG, d)
    ltd = jnp.sqrt(jnp.concatenate(vals, axis=1))
    lidx = jnp.concatenate(idxs, axis=1)
    nbvalid = ltd < BIG * 0.5
    lw = jax.nn.softmax(jnp.where(nbvalid, -ltd, -BIG), axis=-1)
    lw = lw * nbvalid.astype(lw.dtype) * padq[:, None]
    lw_out[0] = lw
    idx_out[0] = jnp.where(nbvalid, lidx, 0) + b * A


def _run_ltopk(leader_pos, pad_f):
    body = functools.partial(_ltopk_body, k=LK)
    out_shapes = [
        jax.ShapeDtypeStruct((B, A, LK), jnp.float32),
        jax.ShapeDtypeStruct((B, A, LK), jnp.int32),
    ]
    spec_q = pl.BlockSpec((1, QBLK, POS), lambda b, i: (b, i, 0))
    spec_all = pl.BlockSpec((1, A, POS), lambda b, i: (b, 0, 0))
    spec_padq = pl.BlockSpec((1, 1, QBLK), lambda b, i: (b, 0, i))
    spec_padall = pl.BlockSpec((1, 1, A), lambda b, i: (b, 0, 0))
    out_spec = pl.BlockSpec((1, QBLK, LK), lambda b, i: (b, i, 0))
    return pl.pallas_call(
        body,
        grid=(B, A // QBLK),
        in_specs=[spec_q, spec_all, spec_padq, spec_padall],
        out_specs=[out_spec, out_spec],
        out_shape=out_shapes,
    )(leader_pos, leader_pos, pad_f[:, None, :], pad_f[:, None, :])


def _stage2_body(opart_ref, rows_ref, memb_ref, pot_ref, npmax_ref, af_ref,
                 tg, tbe, tWf, tbf, iuW, iub, ig, ibe, imW, imb,
                 ew_out, lead_out, h_out, msg_out):
    opart = opart_ref[...]
    rows = rows_ref[...].astype(jnp.float32)
    bpnb = rows[:, :, :TH]
    hidden = opart[:, None, :] + bpnb
    h = jax.nn.gelu(_ln(hidden, tg[...], tbe[...]))
    w2 = tWf[...][:, 0]
    trust = jax.nn.sigmoid(jnp.sum(h * w2[None, None, :], axis=-1) + tbf[...][0, 0])
    ew = memb_ref[...] * trust
    ew = ew / (jnp.sum(ew, axis=-1, keepdims=True) + 1e-8)
    ew_out[...] = ew
    pot = pot_ref[...][:, 0]
    npmax = npmax_ref[...][:, 0]
    lead_out[...] = (pot >= npmax).astype(jnp.float32)[:, None]
    agg = jnp.sum(rows[:, :, TH:2 * TH] * ew[:, :, None], axis=1)
    hn = _ln(af_ref[...] + agg @ iuW[...] + iub[...], ig[...], ibe[...])
    h_out[...] = hn
    msg_out[...] = jax.nn.gelu(hn @ imW[...] + imb[...]).astype(jnp.bfloat16)


def _run_stage2(opart, rows, memb2, pot, npmax2, af2, p):
    tr = p["trust"]
    w_args = [tr["layers"][0]["g"][None], tr["layers"][0]["beta"][None],
              tr["Wf"], tr["bf"][None],
              p["intra_upd_W"], p["intra_upd_b"][None],
              p["intra_g"][None], p["intra_beta"][None],
              p["intra_msg_W"], p["intra_msg_b"][None]]
    in_specs = [
        _row_spec((ROWS, TH), RBLK),
        _row_spec((ROWS, K, TAB1), RBLK),
        _row_spec((ROWS, K), RBLK),
        _row_spec((ROWS, 1), RBLK),
        _row_spec((ROWS, 1), RBLK),
        _row_spec((ROWS, FD), RBLK),
    ] + [_full_spec(w.shape) for w in w_args]
    out_shapes = [
        jax.ShapeDtypeStruct((ROWS, K), jnp.float32),
        jax.ShapeDtypeStruct((ROWS, 1), jnp.float32),
        jax.ShapeDtypeStruct((ROWS, FD), jnp.float32),
        jax.ShapeDtypeStruct((ROWS, MD), jnp.bfloat16),
    ]
    out_specs = [_row_spec(s.shape, RBLK) for s in out_shapes]
    return pl.pallas_call(
        _stage2_body,
        grid=(ROWS // RBLK,),
        in_specs=in_specs,
        out_specs=out_specs,
        out_shape=out_shapes,
    )(opart, rows, memb2, pot, npmax2, af2, *w_args)


def _msg_body(h_ref, Wm, bm, msg_out):
    msg_out[...] = jax.nn.gelu(h_ref[...] @ Wm[...] + bm[...]).astype(jnp.bfloat16)


def _run_msg(h, Wm, bm):
    w_args = [Wm, bm[None]]
    return pl.pallas_call(
        _msg_body,
        grid=(ROWS // RBLK,),
        in_specs=[_row_spec((ROWS, FD), RBLK)] + [_full_spec(w.shape) for w in w_args],
        out_specs=_row_spec((ROWS, MD), RBLK),
        out_shape=jax.ShapeDtypeStruct((ROWS, MD), jnp.bfloat16),
    )(h, *w_args)


def _updmsg_body(h_ref, nb_ref, w_ref, Wu, bu, g, be, Wm, bm, h_out, msg_out):
    nb = nb_ref[...].astype(jnp.float32)
    agg = jnp.sum(nb * w_ref[...][:, :, None], axis=1)
    hn = _ln(h_ref[...] + agg @ Wu[...] + bu[...], g[...], be[...])
    h_out[...] = hn
    msg_out[...] = jax.nn.gelu(hn @ Wm[...] + bm[...]).astype(jnp.bfloat16)


def _run_updmsg(h, nb, w, nk, Wu, bu, g, be, Wm, bm):
    w_args = [Wu, bu[None], g[None], be[None], Wm, bm[None]]
    out_shapes = [
        jax.ShapeDtypeStruct((ROWS, FD), jnp.float32),
        jax.ShapeDtypeStruct((ROWS, MD), jnp.bfloat16),
    ]
    return pl.pallas_call(
        _updmsg_body,
        grid=(ROWS // RBLK,),
        in_specs=[_row_spec((ROWS, FD), RBLK), _row_spec((ROWS, nk, MD), RBLK),
                  _row_spec((ROWS, nk), RBLK)]
        + [_full_spec(w_.shape) for w_ in w_args],
        out_specs=[_row_spec(s.shape, RBLK) for s in out_shapes],
        out_shape=out_shapes,
    )(h, nb, w, *w_args)


def _upd_body(h_ref, nb_ref, w_ref, Wu, bu, g, be, h_out):
    nb = nb_ref[...].astype(jnp.float32)
    agg = jnp.sum(nb * w_ref[...][:, :, None], axis=1)
    hn = _ln(h_ref[...] + agg @ Wu[...] + bu[...], g[...], be[...])
    h_out[...] = hn.astype(jnp.bfloat16)


def _run_upd(h, nb, w, nk, Wu, bu, g, be):
    w_args = [Wu, bu[None], g[None], be[None]]
    return pl.pallas_call(
        _upd_body,
        grid=(ROWS // RBLK,),
        in_specs=[_row_spec((ROWS, FD), RBLK), _row_spec((ROWS, nk, MD), RBLK),
                  _row_spec((ROWS, nk), RBLK)]
        + [_full_spec(w_.shape) for w_ in w_args],
        out_specs=_row_spec((ROWS, FD), RBLK),
        out_shape=jax.ShapeDtypeStruct((ROWS, FD), jnp.bfloat16),
    )(h, nb, w, *w_args)


def _final_body(h_ref, rows_ref, ew_ref, bl_ref,
                iuW, iub, ig, ibe,
                aW0, ab0, ag0, abe0, aW1, ab1, ag1, abe1, aWf, abf,
                cW0, cb0, cg0, cbe0, cW1, cb1, cg1, cbe1, cWf, cbf,
                logits_out, values_out):
    rows = rows_ref[...].astype(jnp.float32)
    ew = ew_ref[...][:, :, None]
    agg = jnp.sum(rows[:, :, :MD] * ew, axis=1)
    hn = _ln(h_ref[...] + agg @ iuW[...] + iub[...], ig[...], ibe[...])
    broadcast = jnp.sum(rows[:, :, MD:] * ew, axis=1)
    x = jnp.concatenate([hn, broadcast, bl_ref[...]], axis=-1)
    a = jax.nn.gelu(_ln(x @ aW0[...] + ab0[...], ag0[...], abe0[...]))
    a = jax.nn.gelu(_ln(a @ aW1[...] + ab1[...], ag1[...], abe1[...]))
    logits_out[...] = a @ aWf[...] + abf[...]
    c = jax.nn.gelu(_ln(x @ cW0[...] + cb0[...], cg0[...], cbe0[...]))
    c = jax.nn.gelu(_ln(c @ cW1[...] + cb1[...], cg1[...], cbe1[...]))
    values_out[...] = c @ cWf[...] + cbf[...]


def _run_final(h, rows, ew, bl, p):
    ac = p["actor"]
    cr = p["critic"]

    def mlp_args(m):
        return [
            m["layers"][0]["W"], m["layers"][0]["b"][None], m["layers"][0]["g"][None], m["layers"][0]["beta"][None],
            m["layers"][1]["W"], m["layers"][1]["b"][None], m["layers"][1]["g"][None], m["layers"][1]["beta"][None],
            m["Wf"], m["bf"][None],
        ]

    w_args = [p["intra_upd_W"], p["intra_upd_b"][None],
              p["intra_g"][None], p["intra_beta"][None]]
    w_args += mlp_args(ac) + mlp_args(cr)
    out_shapes = [
        jax.ShapeDtypeStruct((ROWS, ACT), jnp.float32),
        jax.ShapeDtypeStruct((ROWS, 1), jnp.float32),
    ]
    return pl.pallas_call(
        _final_body,
        grid=(ROWS // RBLK,),
        in_specs=[_row_spec((ROWS, FD), RBLK), _row_spec((ROWS, K, TABF), RBLK),
                  _row_spec((ROWS, K), RBLK), _row_spec((ROWS, BL), RBLK)]
        + [_full_spec(w.shape) for w in w_args],
        out_specs=[_row_spec(s.shape, RBLK) for s in out_shapes],
        out_shape=out_shapes,
    )(h, rows, ew, bl, *w_args)


def kernel(obs, positions, history, params):
    p = params
    obs2 = obs.reshape(ROWS, OBS)
    hist3 = history.reshape(ROWS, HW, BIN)
    pos2 = positions.reshape(ROWS, POS)

    bl2, af2, pot2, opart2, tab1, tab2 = _run_encoder(obs2, hist3, pos2, p)

    gnidx3, membership, npmax3 = _run_topk(positions, pot2)
    gnidx = gnidx3.reshape(-1)

    rows1 = _sc_gather(tab1, gnidx).reshape(ROWS, K, TAB1)
    ew2, lead2, h1, msg2 = _run_stage2(opart2, rows1,
                                       membership.reshape(ROWS, K), pot2,
                                       npmax3.reshape(ROWS, 1), af2, p)
    is_leader = lead2.reshape(B, A) > 0.5

    ar = jnp.arange(A)
    sortkey = jnp.where(is_leader, ar, A + ar)
    order = jnp.argsort(sortkey, axis=-1)
    nlead = is_leader.sum(-1)
    leader_indices = jnp.where(ar[None, :] < nlead[:, None], order, -1)
    pad = leader_indices >= 0
    padf = pad.astype(jnp.float32)

    goffs2 = (jnp.arange(B, dtype=jnp.int32) * A)[:, None]
    tab2z = jnp.concatenate([tab2, jnp.zeros((8, TAB2), jnp.float32)], axis=0)
    lidx = jnp.where(pad, leader_indices + goffs2, ROWS).reshape(-1)
    lrows = _sc_gather(tab2z, lidx)
    leader_feat2 = lrows[:, :FD]
    leader_pos = lrows[:, FD:FD + POS].reshape(B, A, POS)

    lw, glnb = _run_ltopk(leader_pos, padf)
    glnidx = glnb.reshape(-1)
    lw2 = lw.reshape(ROWS, LK)

    hl = leader_feat2
    lmsg = _run_msg(hl, p["inter_msg_W"], p["inter_msg_b"])
    nb = _sc_gather(lmsg, glnidx).reshape(ROWS, LK, MD)
    hl, lmsg = _run_updmsg(hl, nb, lw2, LK, p["inter_upd_W"], p["inter_upd_b"],
                           p["inter_g"], p["inter_beta"],
                           p["inter_msg_W"], p["inter_msg_b"])
    nb = _sc_gather(lmsg, glnidx).reshape(ROWS, LK, MD)
    hl = _run_upd(hl, nb, lw2, LK, p["inter_upd_W"], p["inter_upd_b"],
                  p["inter_g"], p["inter_beta"])

    slots = jnp.broadcast_to(ar[None, :], (B, A))
    scat_idx = jnp.where(pad, leader_indices, A)
    tmp = jnp.full((B, A + 1), -1, dtype=jnp.int32)
    bv = jnp.arange(B)[:, None]
    a2s = tmp.at[bv, scat_idx].set(jnp.where(pad, slots, -1).astype(jnp.int32))[:, :A]
    tableHL = jnp.concatenate([hl, jnp.zeros((8, FD), jnp.bfloat16)], axis=0)
    gidx_a2s = jnp.where(a2s >= 0, a2s + goffs2, ROWS).reshape(-1)
    gtab = _sc_gather(tableHL, gidx_a2s)

    tabF = jnp.concatenate([msg2, gtab], axis=-1)
    rowsF = _sc_gather(tabF, gnidx).reshape(ROWS, K, TABF)

    logits2, values2 = _run_final(h1, rowsF, ew2, bl2, p)
    return logits2.reshape(B, A, ACT), values2.reshape(B, A)


# speedup vs baseline: 1.0347x; 1.0347x over previous
import functools

import jax
import jax.numpy as jnp
from jax import lax
from jax.experimental import pallas as pl
from jax.experimental.pallas import tpu as pltpu
from jax.experimental.pallas import tpu_sc as plsc

B = 4
A = 2048
OBS = 64
POS = 3
K = 16
LK = 8
HW = 16
BIN = 16
OBS_H = 128
OBS_E = 64
BH = 64
BL = 32
TH = 64
FD = 64
LH = 64
MD = 64
INTRA = 2
INTER = 2
PH = 128
VH = 128
ACT = 32
BIG = 1e9

ROWS = B * A
RBLK = 512
QBLK = 256
TAB1 = 128
TAB2 = 80
TABF = 128


def _ln(x, g, b):
    m = x.mean(-1, keepdims=True)
    v = ((x - m) ** 2).mean(-1, keepdims=True)
    return (x - m) / jnp.sqrt(v + 1e-5) * g + b


def _full_spec(shape):
    n = len(shape)
    return pl.BlockSpec(shape, lambda *_: (0,) * n)


def _row_spec(shape, blk):
    rest = shape[1:]
    n = len(shape)
    return pl.BlockSpec((blk,) + rest, lambda i: (i,) + (0,) * (n - 1))


_SC_NC = 2
_SC_NS = 16
_SC_NW = _SC_NC * _SC_NS


@functools.lru_cache(maxsize=None)
def _make_sc_gather(M, N, D, dtype):
    b_per_w = M // _SC_NW
    row_bytes = D * dtype.itemsize
    chunk = min(max(98304 // row_bytes // 256 * 256, 256), b_per_w)
    nch = b_per_w // chunk
    nbuf = min(4, nch)
    mesh = plsc.VectorSubcoreMesh(core_axis_name="c", subcore_axis_name="s")

    @functools.partial(
        pl.kernel,
        mesh=mesh,
        compiler_params=pltpu.CompilerParams(use_tc_tiling_on_sc=False),
        out_type=jax.ShapeDtypeStruct((M, D), dtype),
        scratch_types=[pltpu.VMEM((b_per_w,), jnp.int32)]
        + [pltpu.VMEM((chunk, D), dtype) for _ in range(nbuf)]
        + [pltpu.SemaphoreType.DMA for _ in range(nbuf)],
    )
    def gather_k(table_hbm, idx_hbm, out_hbm, idx_v, *bufsem):
        bufs = bufsem[:nbuf]
        sems = bufsem[nbuf:]
        wid = lax.axis_index("s") * _SC_NC + lax.axis_index("c")
        base = wid * b_per_w
        pltpu.sync_copy(idx_hbm.at[pl.ds(base, b_per_w)], idx_v)
        cps = [None] * nbuf
        for j in range(nbuf):
            cps[j] = pltpu.async_copy(
                table_hbm.at[idx_v.at[pl.ds(j * chunk, chunk)]], bufs[j], sems[j])
        for c in range(nch):
            j = c % nbuf
            cps[j].wait()
            pltpu.sync_copy(bufs[j], out_hbm.at[pl.ds(base + c * chunk, chunk)])
            nxt = c + nbuf
            if nxt < nch:
                cps[j] = pltpu.async_copy(
                    table_hbm.at[idx_v.at[pl.ds(nxt * chunk, chunk)]],
                    bufs[j], sems[j])

    return gather_k


def _sc_gather(table, idx_flat):
    N, D = table.shape
    (M,) = idx_flat.shape
    return _make_sc_gather(M, N, D, table.dtype)(table, idx_flat)


def _encoder_body(obs_ref, hist_ref, pos_ref,
                  bW0, bb0, bg0, bbe0, bWf, bbf,
                  oW0, ob0, og0, obe0, oW1, ob1, og1, obe1, oWf, obf,
                  fW0, fb0, fg0, fbe0, fWf, fbf,
                  pW0, pb0, pg0, pbe0, pWf, pbf,
                  tW1a, tW1b, tb1, imW, imb,
                  bl_out, af_out, pot_out, opart_out, tab1_out, tab2_out):
    hist = hist_ref[...]
    hmean = jnp.mean(hist, axis=1)
    x = jax.nn.gelu(_ln(hmean @ bW0[...] + bb0[...], bg0[...], bbe0[...]))
    bl = x @ bWf[...] + bbf[...]

    o = obs_ref[...]
    o = jax.nn.gelu(_ln(o @ oW0[...] + ob0[...], og0[...], obe0[...]))
    o = jax.nn.gelu(_ln(o @ oW1[...] + ob1[...], og1[...], obe1[...]))
    oe = o @ oWf[...] + obf[...]

    f_in = jnp.concatenate([oe, bl], axis=-1)
    f = jax.nn.gelu(_ln(f_in @ fW0[...] + fb0[...], fg0[...], fbe0[...]))
    af = f @ fWf[...] + fbf[...]

    p = jax.nn.gelu(_ln(af @ pW0[...] + pb0[...], pg0[...], pbe0[...]))
    pot = p @ pWf[...] + pbf[...]

    bp = bl @ tW1b[...]
    msg1 = jax.nn.gelu(af @ imW[...] + imb[...])
    blk = af.shape[0]
    zeros2 = jnp.zeros((blk, TAB2 - FD - POS), jnp.float32)

    bl_out[...] = bl
    af_out[...] = af
    pot_out[...] = pot
    opart_out[...] = oe @ tW1a[...] + tb1[...]
    tab1_out[...] = jnp.concatenate([bp, msg1], axis=-1).astype(jnp.bfloat16)
    tab2_out[...] = jnp.concatenate([af, pos_ref[...], zeros2], axis=-1)


def _run_encoder(obs2, hist3, pos2, p):
    be = p["behavior"]
    ob = p["obs_encoder"]
    fp = p["feature_proj"]
    lp = p["leader_pot"]
    tr = p["trust"]
    tW1 = tr["layers"][0]["W"]
    w_args = [
        be["layers"][0]["W"], be["layers"][0]["b"][None], be["layers"][0]["g"][None], be["layers"][0]["beta"][None],
        be["Wf"], be["bf"][None],
        ob["layers"][0]["W"], ob["layers"][0]["b"][None], ob["layers"][0]["g"][None], ob["layers"][0]["beta"][None],
        ob["layers"][1]["W"], ob["layers"][1]["b"][None], ob["layers"][1]["g"][None], ob["layers"][1]["beta"][None],
        ob["Wf"], ob["bf"][None],
        fp["layers"][0]["W"], fp["layers"][0]["b"][None], fp["layers"][0]["g"][None], fp["layers"][0]["beta"][None],
        fp["Wf"], fp["bf"][None],
        lp["layers"][0]["W"], lp["layers"][0]["b"][None], lp["layers"][0]["g"][None], lp["layers"][0]["beta"][None],
        lp["Wf"], lp["bf"][None],
        tW1[:OBS_E], tW1[OBS_E:], tr["layers"][0]["b"][None],
        p["intra_msg_W"], p["intra_msg_b"][None],
    ]
    in_specs = [_row_spec((ROWS, OBS), RBLK), _row_spec((ROWS, HW, BIN), RBLK),
                _row_spec((ROWS, POS), RBLK)]
    in_specs += [_full_spec(w.shape) for w in w_args]
    out_shapes = [
        jax.ShapeDtypeStruct((ROWS, BL), jnp.float32),
        jax.ShapeDtypeStruct((ROWS, FD), jnp.float32),
        jax.ShapeDtypeStruct((ROWS, 1), jnp.float32),
        jax.ShapeDtypeStruct((ROWS, TH), jnp.float32),
        jax.ShapeDtypeStruct((ROWS, TAB1), jnp.bfloat16),
        jax.ShapeDtypeStruct((ROWS, TAB2), jnp.float32),
    ]
    out_specs = [_row_spec(s.shape, RBLK) for s in out_shapes]
    return pl.pallas_call(
        _encoder_body,
        grid=(ROWS // RBLK,),
        in_specs=in_specs,
        out_specs=out_specs,
        out_shape=out_shapes,
    )(obs2, hist3, pos2, *w_args)


def _topk_body(pq_ref, pall_ref, pot_ref, idx_out, memb_out, npmax_out, *, k):
    pq = pq_ref[0]
    pall = pall_ref[0]
    potrow = pot_ref[0]
    sq_q = jnp.sum(pq * pq, axis=-1, keepdims=True)
    sq_a = jnp.sum(pall * pall, axis=-1)[None, :]
    cross = jax.lax.dot_general(pq, pall, (((1,), (1,)), ((), ())),
                                preferred_element_type=jnp.float32)
    d2 = sq_q + sq_a - 2.0 * cross
    d = jnp.clip(d2, 0.0) + 1e-12
    b = pl.program_id(0)
    i = pl.program_id(1)
    rowid = i * pq.shape[0] + jax.lax.broadcasted_iota(jnp.int32, d.shape, 0)
    colid = jax.lax.broadcasted_iota(jnp.int32, d.shape, 1)
    d = jnp.where(rowid == colid, BIG * BIG, d)
    vals = []
    idxs = []
    selmask = jnp.zeros(d.shape, jnp.bool_)
    for _ in range(k):
        mval = jnp.min(d, axis=1, keepdims=True)
        sel = jnp.min(jnp.where(d == mval, colid, A), axis=1, keepdims=True)
        vals.append(mval)
        idxs.append(sel)
        hit = colid == sel
        selmask = selmask | hit
        d = jnp.where(hit, BIG * BIG, d)
    dvals = jnp.sqrt(jnp.concatenate(vals, axis=1))
    npmax = jnp.max(jnp.where(selmask, potrow, -BIG), axis=1, keepdims=True)
    idx_out[0] = jnp.concatenate(idxs, axis=1) + b * A
    memb_out[0] = jax.nn.softmax(-dvals / 1.0, axis=-1)
    npmax_out[0] = npmax


def _run_topk(positions, pot2):
    body = functools.partial(_topk_body, k=K)
    out_shapes = [
        jax.ShapeDtypeStruct((B, A, K), jnp.int32),
        jax.ShapeDtypeStruct((B, A, K), jnp.float32),
        jax.ShapeDtypeStruct((B, A, 1), jnp.float32),
    ]
    spec_q = pl.BlockSpec((1, QBLK, POS), lambda b, i: (b, i, 0))
    spec_all = pl.BlockSpec((1, A, POS), lambda b, i: (b, 0, 0))
    spec_pot = pl.BlockSpec((1, 1, A), lambda b, i: (b, 0, 0))
    out_spec = pl.BlockSpec((1, QBLK, K), lambda b, i: (b, i, 0))
    out_spec_n = pl.BlockSpec((1, QBLK, 1), lambda b, i: (b, i, 0))
    return pl.pallas_call(
        body,
        grid=(B, A // QBLK),
        in_specs=[spec_q, spec_all, spec_pot],
        out_specs=[out_spec, out_spec, out_spec_n],
        out_shape=out_shapes,
    )(positions, positions, pot2.reshape(B, 1, A))


def _ltopk_body(pq_ref, pall_ref, padq_ref, padall_ref, lw_out, idx_out, *, k):
    pq = pq_ref[0]
    pall = pall_ref[0]
    padq = padq_ref[0][0]
    padall = padall_ref[0][0]
    sq_q = jnp.sum(pq * pq, axis=-1, keepdims=True)
    sq_a = jnp.sum(pall * pall, axis=-1)[None, :]
    cross = jax.lax.dot_general(pq, pall, (((1,), (1,)), ((), ())),
                                preferred_element_type=jnp.float32)
    d2 = sq_q + sq_a - 2.0 * cross
    d = jnp.clip(d2, 0.0) + 1e-12
    b = pl.program_id(0)
    i = pl.program_id(1)
    rowid = i * pq.shape[0] + jax.lax.broadcasted_iota(jnp.int32, d.shape, 0)
    colid = jax.lax.broadcasted_iota(jnp.int32, d.shape, 1)
    valid = (padq[:, None] > 0.5) & (padall[None, :] > 0.5) & (rowid != colid)
    d = jnp.where(valid, d, BIG * BIG)
    vals = []
    idxs = []
    for _ in range(k):
        mval = jnp.min(d, axis=1, keepdims=True)
        sel = jnp.min(jnp.where(d == mval, colid, A), axis=1, keepdims=True)
        vals.append(mval)
        idxs.append(sel)
        d = jnp.where(colid == sel, BIG * BIG, d)
    ltd = jnp.sqrt(jnp.concatenate(vals, axis=1))
    lidx = jnp.concatenate(idxs, axis=1)
    nbvalid = ltd < BIG * 0.5
    lw = jax.nn.softmax(jnp.where(nbvalid, -ltd, -BIG), axis=-1)
    lw = lw * nbvalid.astype(lw.dtype) * padq[:, None]
    lw_out[0] = lw
    idx_out[0] = jnp.where(nbvalid, lidx, 0) + b * A


def _run_ltopk(leader_pos, pad_f):
    body = functools.partial(_ltopk_body, k=LK)
    out_shapes = [
        jax.ShapeDtypeStruct((B, A, LK), jnp.float32),
        jax.ShapeDtypeStruct((B, A, LK), jnp.int32),
    ]
    spec_q = pl.BlockSpec((1, QBLK, POS), lambda b, i: (b, i, 0))
    spec_all = pl.BlockSpec((1, A, POS), lambda b, i: (b, 0, 0))
    spec_padq = pl.BlockSpec((1, 1, QBLK), lambda b, i: (b, 0, i))
    spec_padall = pl.BlockSpec((1, 1, A), lambda b, i: (b, 0, 0))
    out_spec = pl.BlockSpec((1, QBLK, LK), lambda b, i: (b, i, 0))
    return pl.pallas_call(
        body,
        grid=(B, A // QBLK),
        in_specs=[spec_q, spec_all, spec_padq, spec_padall],
        out_specs=[out_spec, out_spec],
        out_shape=out_shapes,
    )(leader_pos, leader_pos, pad_f[:, None, :], pad_f[:, None, :])


def _stage2_body(opart_ref, rows_ref, memb_ref, pot_ref, npmax_ref, af_ref,
                 tg, tbe, tWf, tbf, iuW, iub, ig, ibe, imW, imb,
                 ew_out, lead_out, h_out, msg_out):
    opart = opart_ref[...]
    rows = rows_ref[...].astype(jnp.float32)
    bpnb = rows[:, :, :TH]
    hidden = opart[:, None, :] + bpnb
    h = jax.nn.gelu(_ln(hidden, tg[...], tbe[...]))
    w2 = tWf[...][:, 0]
    trust = jax.nn.sigmoid(jnp.sum(h * w2[None, None, :], axis=-1) + tbf[...][0, 0])
    ew = memb_ref[...] * trust
    ew = ew / (jnp.sum(ew, axis=-1, keepdims=True) + 1e-8)
    ew_out[...] = ew
    pot = pot_ref[...][:, 0]
    npmax = npmax_ref[...][:, 0]
    lead_out[...] = (pot >= npmax).astype(jnp.float32)[:, None]
    agg = jnp.sum(rows[:, :, TH:2 * TH] * ew[:, :, None], axis=1)
    hn = _ln(af_ref[...] + agg @ iuW[...] + iub[...], ig[...], ibe[...])
    h_out[...] = hn
    msg_out[...] = jax.nn.gelu(hn @ imW[...] + imb[...]).astype(jnp.bfloat16)


def _run_stage2(opart, rows, memb2, pot, npmax2, af2, p):
    tr = p["trust"]
    w_args = [tr["layers"][0]["g"][None], tr["layers"][0]["beta"][None],
              tr["Wf"], tr["bf"][None],
              p["intra_upd_W"], p["intra_upd_b"][None],
              p["intra_g"][None], p["intra_beta"][None],
              p["intra_msg_W"], p["intra_msg_b"][None]]
    in_specs = [
        _row_spec((ROWS, TH), RBLK),
        _row_spec((ROWS, K, TAB1), RBLK),
        _row_spec((ROWS, K), RBLK),
        _row_spec((ROWS, 1), RBLK),
        _row_spec((ROWS, 1), RBLK),
        _row_spec((ROWS, FD), RBLK),
    ] + [_full_spec(w.shape) for w in w_args]
    out_shapes = [
        jax.ShapeDtypeStruct((ROWS, K), jnp.float32),
        jax.ShapeDtypeStruct((ROWS, 1), jnp.float32),
        jax.ShapeDtypeStruct((ROWS, FD), jnp.float32),
        jax.ShapeDtypeStruct((ROWS, MD), jnp.bfloat16),
    ]
    out_specs = [_row_spec(s.shape, RBLK) for s in out_shapes]
    return pl.pallas_call(
        _stage2_body,
        grid=(ROWS // RBLK,),
        in_specs=in_specs,
        out_specs=out_specs,
        out_shape=out_shapes,
    )(opart, rows, memb2, pot, npmax2, af2, *w_args)


def _msg_body(h_ref, Wm, bm, msg_out):
    msg_out[...] = jax.nn.gelu(h_ref[...] @ Wm[...] + bm[...]).astype(jnp.bfloat16)


def _run_msg(h, Wm, bm):
    w_args = [Wm, bm[None]]
    return pl.pallas_call(
        _msg_body,
        grid=(ROWS // RBLK,),
        in_specs=[_row_spec((ROWS, FD), RBLK)] + [_full_spec(w.shape) for w in w_args],
        out_specs=_row_spec((ROWS, MD), RBLK),
        out_shape=jax.ShapeDtypeStruct((ROWS, MD), jnp.bfloat16),
    )(h, *w_args)


def _updmsg_body(h_ref, nb_ref, w_ref, Wu, bu, g, be, Wm, bm, h_out, msg_out):
    nb = nb_ref[...].astype(jnp.float32)
    agg = jnp.sum(nb * w_ref[...][:, :, None], axis=1)
    hn = _ln(h_ref[...] + agg @ Wu[...] + bu[...], g[...], be[...])
    h_out[...] = hn
    msg_out[...] = jax.nn.gelu(hn @ Wm[...] + bm[...]).astype(jnp.bfloat16)


def _run_updmsg(h, nb, w, nk, Wu, bu, g, be, Wm, bm):
    w_args = [Wu, bu[None], g[None], be[None], Wm, bm[None]]
    out_shapes = [
        jax.ShapeDtypeStruct((ROWS, FD), jnp.float32),
        jax.ShapeDtypeStruct((ROWS, MD), jnp.bfloat16),
    ]
    return pl.pallas_call(
        _updmsg_body,
        grid=(ROWS // RBLK,),
        in_specs=[_row_spec((ROWS, FD), RBLK), _row_spec((ROWS, nk, MD), RBLK),
                  _row_spec((ROWS, nk), RBLK)]
        + [_full_spec(w_.shape) for w_ in w_args],
        out_specs=[_row_spec(s.shape, RBLK) for s in out_shapes],
        out_shape=out_shapes,
    )(h, nb, w, *w_args)


def _upd_body(h_ref, nb_ref, w_ref, Wu, bu, g, be, h_out):
    nb = nb_ref[...].astype(jnp.float32)
    agg = jnp.sum(nb * w_ref[...][:, :, None], axis=1)
    hn = _ln(h_ref[...] + agg @ Wu[...] + bu[...], g[...], be[...])
    h_out[...] = hn.astype(jnp.bfloat16)


def _run_upd(h, nb, w, nk, Wu, bu, g, be):
    w_args = [Wu, bu[None], g[None], be[None]]
    return pl.pallas_call(
        _upd_body,
        grid=(ROWS // RBLK,),
        in_specs=[_row_spec((ROWS, FD), RBLK), _row_spec((ROWS, nk, MD), RBLK),
                  _row_spec((ROWS, nk), RBLK)]
        + [_full_spec(w_.shape) for w_ in w_args],
        out_specs=_row_spec((ROWS, FD), RBLK),
        out_shape=jax.ShapeDtypeStruct((ROWS, FD), jnp.bfloat16),
    )(h, nb, w, *w_args)


def _final_body(h_ref, rows_ref, ew_ref, bl_ref,
                iuW, iub, ig, ibe,
                aW0, ab0, ag0, abe0, aW1, ab1, ag1, abe1, aWf, abf,
                cW0, cb0, cg0, cbe0, cW1, cb1, cg1, cbe1, cWf, cbf,
                logits_out, values_out):
    rows = rows_ref[...].astype(jnp.float32)
    ew = ew_ref[...][:, :, None]
    agg = jnp.sum(rows[:, :, :MD] * ew, axis=1)
    hn = _ln(h_ref[...] + agg @ iuW[...] + iub[...], ig[...], ibe[...])
    broadcast = jnp.sum(rows[:, :, MD:] * ew, axis=1)
    x = jnp.concatenate([hn, broadcast, bl_ref[...]], axis=-1)
    a = jax.nn.gelu(_ln(x @ aW0[...] + ab0[...], ag0[...], abe0[...]))
    a = jax.nn.gelu(_ln(a @ aW1[...] + ab1[...], ag1[...], abe1[...]))
    logits_out[...] = a @ aWf[...] + abf[...]
    c = jax.nn.gelu(_ln(x @ cW0[...] + cb0[...], cg0[...], cbe0[...]))
    c = jax.nn.gelu(_ln(c @ cW1[...] + cb1[...], cg1[...], cbe1[...]))
    values_out[...] = c @ cWf[...] + cbf[...]


def _run_final(h, rows, ew, bl, p):
    ac = p["actor"]
    cr = p["critic"]

    def mlp_args(m):
        return [
            m["layers"][0]["W"], m["layers"][0]["b"][None], m["layers"][0]["g"][None], m["layers"][0]["beta"][None],
            m["layers"][1]["W"], m["layers"][1]["b"][None], m["layers"][1]["g"][None], m["layers"][1]["beta"][None],
            m["Wf"], m["bf"][None],
        ]

    w_args = [p["intra_upd_W"], p["intra_upd_b"][None],
              p["intra_g"][None], p["intra_beta"][None]]
    w_args += mlp_args(ac) + mlp_args(cr)
    out_shapes = [
        jax.ShapeDtypeStruct((ROWS, ACT), jnp.float32),
        jax.ShapeDtypeStruct((ROWS, 1), jnp.float32),
    ]
    return pl.pallas_call(
        _final_body,
        grid=(ROWS // RBLK,),
        in_specs=[_row_spec((ROWS, FD), RBLK), _row_spec((ROWS, K, TABF), RBLK),
                  _row_spec((ROWS, K), RBLK), _row_spec((ROWS, BL), RBLK)]
        + [_full_spec(w.shape) for w in w_args],
        out_specs=[_row_spec(s.shape, RBLK) for s in out_shapes],
        out_shape=out_shapes,
    )(h, rows, ew, bl, *w_args)


def kernel(obs, positions, history, params):
    p = params
    obs2 = obs.reshape(ROWS, OBS)
    hist3 = history.reshape(ROWS, HW, BIN)
    pos2 = positions.reshape(ROWS, POS)

    bl2, af2, pot2, opart2, tab1, tab2 = _run_encoder(obs2, hist3, pos2, p)

    gnidx3, membership, npmax3 = _run_topk(positions, pot2)
    gnidx = gnidx3.reshape(-1)

    rows1 = _sc_gather(tab1, gnidx).reshape(ROWS, K, TAB1)
    ew2, lead2, h1, msg2 = _run_stage2(opart2, rows1,
                                       membership.reshape(ROWS, K), pot2,
                                       npmax3.reshape(ROWS, 1), af2, p)
    is_leader = lead2.reshape(B, A) > 0.5

    ar = jnp.arange(A)
    sortkey = jnp.where(is_leader, ar, A + ar)
    order = jnp.argsort(sortkey, axis=-1)
    nlead = is_leader.sum(-1)
    leader_indices = jnp.where(ar[None, :] < nlead[:, None], order, -1)
    pad = leader_indices >= 0
    padf = pad.astype(jnp.float32)

    goffs2 = (jnp.arange(B, dtype=jnp.int32) * A)[:, None]
    tab2z = jnp.concatenate([tab2, jnp.zeros((8, TAB2), jnp.float32)], axis=0)
    lidx = jnp.where(pad, leader_indices + goffs2, ROWS).reshape(-1)
    lrows = _sc_gather(tab2z, lidx)
    leader_feat2 = lrows[:, :FD]
    leader_pos = lrows[:, FD:FD + POS].reshape(B, A, POS)

    lw, glnb = _run_ltopk(leader_pos, padf)
    glnidx = glnb.reshape(-1)
    lw2 = lw.reshape(ROWS, LK)

    hl = leader_feat2
    lmsg = _run_msg(hl, p["inter_msg_W"], p["inter_msg_b"])
    nb = _sc_gather(lmsg, glnidx).reshape(ROWS, LK, MD)
    hl, lmsg = _run_updmsg(hl, nb, lw2, LK, p["inter_upd_W"], p["inter_upd_b"],
                           p["inter_g"], p["inter_beta"],
                           p["inter_msg_W"], p["inter_msg_b"])
    nb = _sc_gather(lmsg, glnidx).reshape(ROWS, LK, MD)
    hl = _run_upd(hl, nb, lw2, LK, p["inter_upd_W"], p["inter_upd_b"],
                  p["inter_g"], p["inter_beta"])

    slots = jnp.broadcast_to(ar[None, :], (B, A))
    scat_idx = jnp.where(pad, leader_indices, A)
    tmp = jnp.full((B, A + 1), -1, dtype=jnp.int32)
    bv = jnp.arange(B)[:, None]
    a2s = tmp.at[bv, scat_idx].set(jnp.where(pad, slots, -1).astype(jnp.int32))[:, :A]
    tableHL = jnp.concatenate([hl, jnp.zeros((8, FD), jnp.bfloat16)], axis=0)
    gidx_a2s = jnp.where(a2s >= 0, a2s + goffs2, ROWS).reshape(-1)
    gtab = _sc_gather(tableHL, gidx_a2s)

    tabF = jnp.concatenate([msg2, gtab], axis=-1)
    rowsF = _sc_gather(tabF, gnidx).reshape(ROWS, K, TABF)

    logits2, values2 = _run_final(h1, rowsF, ew2, bl2, p)
    return logits2.reshape(B, A, ACT), values2.reshape(B, A)
